# baseline (speedup 1.0000x reference)
import functools

import jax
import jax.numpy as jnp
from jax import lax
from jax.experimental import pallas as pl
from jax.experimental.pallas import tpu as pltpu
from jax.experimental.pallas import tpu_sc as plsc

E = 64
TOPK = 2
H = 768
I_EXP = 256
I_SH = 1536
T = 2048
CAP = 256
NSLOT = E * CAP
TRASH = NSLOT
IDXBUF = NSLOT + 16
DUMMY_ROW = T

NC, NS = 2, 16
NW = NC * NS


def _sigmoid(x):
    return 1.0 / (1.0 + jnp.exp(-x))


def _silu(x):
    return x * _sigmoid(x)


def _router_body(x_ref, gw_ref, logits_ref, ds_ref, dg_ref, w_ref, c_ref, oh_ref):
    x = x_ref[...]
    gw = gw_ref[...]
    logits = lax.dot_general(x, gw, (((1,), (1,)), ((), ())),
                             preferred_element_type=jnp.float32)
    logits_ref[...] = logits
    m = jnp.max(logits, axis=1, keepdims=True)
    ex = jnp.exp(logits - m)
    rw = ex / jnp.sum(ex, axis=1, keepdims=True)

    ii = lax.broadcasted_iota(jnp.int32, (T, E), 1)
    m1 = jnp.max(rw, axis=1, keepdims=True)
    a1 = jnp.min(jnp.where(rw == m1, ii, E), axis=1, keepdims=True)
    rw2 = jnp.where(ii == a1, -1.0, rw)
    m2 = jnp.max(rw2, axis=1, keepdims=True)
    a2 = jnp.min(jnp.where(rw2 == m2, ii, E), axis=1, keepdims=True)

    denom = jnp.maximum(m1 + m2, 1e-6)
    w0 = m1 / denom
    w1 = m2 / denom

    oh1 = (ii == a1).astype(jnp.float32)
    oh2 = (ii == a2).astype(jnp.float32)
    oh_ref[...] = oh1 + oh2

    r = lax.broadcasted_iota(jnp.int32, (256, 256), 0)
    c = lax.broadcasted_iota(jnp.int32, (256, 256), 1)
    tri = (c < r).astype(jnp.float32)

    def blk(i, carry):
        ohb = oh_ref[pl.ds(i * 256, 256), :]
        cb = lax.dot_general(tri, ohb, (((1,), (0,)), ((), ())),
                             preferred_element_type=jnp.float32)
        c_ref[pl.ds(i * 256, 256), :] = cb + carry
        return carry + jnp.sum(ohb, axis=0, keepdims=True)

    lax.fori_loop(0, 8, blk, jnp.zeros((1, E), jnp.float32))

    cexcl = c_ref[...]
    pos0 = jnp.sum(cexcl * oh1, axis=1, keepdims=True).astype(jnp.int32)
    pos1 = jnp.sum(cexcl * oh2, axis=1, keepdims=True).astype(jnp.int32)

    d0 = a1 * CAP + pos0
    d1 = a2 * CAP + pos1
    v0 = pos0 < CAP
    v1 = pos1 < CAP
    ds_ref[...] = jnp.concatenate(
        [jnp.where(v0, d0, TRASH), jnp.where(v1, d1, TRASH)], axis=1)
    dg_ref[...] = jnp.concatenate(
        [jnp.where(v0, d0, 0), jnp.where(v1, d1, 0)], axis=1)
    w_ref[...] = jnp.concatenate(
        [jnp.where(v0, w0, 0.0), jnp.where(v1, w1, 0.0)], axis=1)


def _router(x, gate_w):
    return pl.pallas_call(
        _router_body,
        out_shape=(
            jax.ShapeDtypeStruct((T, E), jnp.float32),
            jax.ShapeDtypeStruct((T, TOPK), jnp.int32),
            jax.ShapeDtypeStruct((T, TOPK), jnp.int32),
            jax.ShapeDtypeStruct((T, TOPK), jnp.float32),
        ),
        scratch_shapes=[pltpu.VMEM((T, E), jnp.float32),
                        pltpu.VMEM((T, E), jnp.float32)],
    )(x, gate_w)


def _scatter_body(ds_hbm, init_hbm, out_hbm, idxv, dstv, sem):
    ci = lax.axis_index("c")
    si = lax.axis_index("s")

    @pl.when(jnp.logical_and(ci == 0, si == 0))
    def _():
        pltpu.sync_copy(init_hbm, idxv)
        pltpu.sync_copy(ds_hbm, dstv)
        lanes = lax.iota(jnp.int32, 16)

        def step(j, _):
            idx = dstv[pl.ds(j * 16, 16)]
            tok = (j * 16 + lanes) >> 1
            plsc.store_scatter(idxv, [idx], tok)
            return 0

        lax.fori_loop(0, (T * TOPK) // 16, step, 0)
        pltpu.sync_copy(idxv, out_hbm)


def _scatter(ds_flat, idx_init):
    k = functools.partial(
        pl.kernel,
        out_type=jax.ShapeDtypeStruct((IDXBUF,), jnp.int32),
        mesh=plsc.VectorSubcoreMesh(core_axis_name="c", subcore_axis_name="s"),
        scratch_types=[
            pltpu.VMEM((IDXBUF,), jnp.int32),
            pltpu.VMEM((T * TOPK,), jnp.int32),
            pltpu.SemaphoreType.DMA,
        ],
        compiler_params=pltpu.CompilerParams(needs_layout_passes=False),
    )(_scatter_body)
    return k(ds_flat, idx_init)


def _gather_rows_body(idx_hbm, tab_hbm, out_hbm, idxv, rows, sem, *,
                      nrows, chunk, idx_off):
    ci = lax.axis_index("c")
    si = lax.axis_index("s")
    wid = si * NC + ci
    per_w = nrows // NW
    base = wid * per_w

    def step(ch, _):
        off = base + ch * chunk
        pltpu.sync_copy(idx_hbm.at[pl.ds(idx_off + off, chunk)], idxv)
        pltpu.async_copy(tab_hbm.at[idxv], rows, sem).wait()
        pltpu.sync_copy(rows, out_hbm.at[pl.ds(off, chunk)])
        return 0

    lax.fori_loop(0, per_w // chunk, step, 0)


def _gather_rows(idx, table, nrows, chunk=128, idx_off=0):
    body = functools.partial(_gather_rows_body, nrows=nrows, chunk=chunk,
                             idx_off=idx_off)
    k = functools.partial(
        pl.kernel,
        out_type=jax.ShapeDtypeStruct((nrows, H), jnp.float32),
        mesh=plsc.VectorSubcoreMesh(core_axis_name="c", subcore_axis_name="s"),
        scratch_types=[
            pltpu.VMEM((chunk,), jnp.int32),
            pltpu.VMEM((chunk, H), jnp.float32),
            pltpu.SemaphoreType.DMA,
        ],
    )(body)
    return k(idx, table)


NG = 2
EG = E // NG


def _experts_body(xg_ref, eg_ref, eu_ref, ed_ref, *rest):
    out_ref = rest[-1]
    cur = xg_ref[...]
    g = lax.dot_general(cur, eg_ref[0], (((1,), (1,)), ((), ())),
                        preferred_element_type=jnp.float32)
    u = lax.dot_general(cur, eu_ref[0], (((1,), (1,)), ((), ())),
                        preferred_element_type=jnp.float32)
    h = _silu(g) * u
    out_ref[...] = lax.dot_general(h, ed_ref[0], (((1,), (1,)), ((), ())),
                                   preferred_element_type=jnp.float32)


def _experts_group(xg_g, eg, eu, ed, prev, g):
    in_specs = [
        pl.BlockSpec((CAP, H), lambda e: (e, 0)),
        pl.BlockSpec((1, I_EXP, H), lambda e: (g * EG + e, 0, 0)),
        pl.BlockSpec((1, I_EXP, H), lambda e: (g * EG + e, 0, 0)),
        pl.BlockSpec((1, H, I_EXP), lambda e: (g * EG + e, 0, 0)),
    ]
    args = [xg_g, eg, eu, ed]
    aliases = {}
    if prev is not None:
        in_specs.append(pl.BlockSpec(memory_space=pltpu.MemorySpace.HBM))
        args.append(prev)
        aliases = {4: 0}
    return pl.pallas_call(
        _experts_body,
        grid=(EG,),
        in_specs=in_specs,
        out_specs=pl.BlockSpec((CAP, H), lambda e: (g * EG + e, 0)),
        out_shape=jax.ShapeDtypeStruct((NSLOT, H), jnp.float32),
        input_output_aliases=aliases,
    )(*args)


def _final_body(x_ref, m0_ref, m1_ref, w_ref, sg_ref, su_ref, sd_ref, seg_ref,
                out_ref):
    xb = x_ref[...]
    wv = w_ref[...]
    moe_sum = m0_ref[...] * wv[:, 0:1] + m1_ref[...] * wv[:, 1:2]

    g = lax.dot_general(xb, sg_ref[...], (((1,), (1,)), ((), ())),
                        preferred_element_type=jnp.float32)
    u = lax.dot_general(xb, su_ref[...], (((1,), (1,)), ((), ())),
                        preferred_element_type=jnp.float32)
    s = lax.dot_general(_silu(g) * u, sd_ref[...], (((1,), (1,)), ((), ())),
                        preferred_element_type=jnp.float32)
    gate = _sigmoid(lax.dot_general(xb, seg_ref[...], (((1,), (1,)), ((), ())),
                                    preferred_element_type=jnp.float32))
    out_ref[...] = moe_sum + gate * s


def _final(x, moe, w01, sgw, suw, sdw, segw):
    nblk = T // 256
    return pl.pallas_call(
        _final_body,
        grid=(nblk,),
        in_specs=[
            pl.BlockSpec((256, H), lambda i: (i, 0)),
            pl.BlockSpec((256, H), lambda i: (i, 0)),
            pl.BlockSpec((256, H), lambda i: (i + nblk, 0)),
            pl.BlockSpec((256, TOPK), lambda i: (i, 0)),
            pl.BlockSpec((I_SH, H), lambda i: (0, 0)),
            pl.BlockSpec((I_SH, H), lambda i: (0, 0)),
            pl.BlockSpec((H, I_SH), lambda i: (0, 0)),
            pl.BlockSpec((1, H), lambda i: (0, 0)),
        ],
        out_specs=pl.BlockSpec((256, H), lambda i: (i, 0)),
        out_shape=jax.ShapeDtypeStruct((T, H), jnp.float32),
    )(x, moe, moe, w01, sgw, suw, sdw, segw)


def kernel(hidden_states, gate_w, expert_gate_w, expert_up_w, expert_down_w,
           shared_gate_w, shared_up_w, shared_down_w, shared_expert_gate_w):
    Bsz, Sl, Hd = hidden_states.shape
    x = hidden_states.reshape(T, H)

    logits, d_s, d_g, w01 = _router(x, gate_w)

    idx_init = lax.iota(jnp.int32, IDXBUF) % T
    idxbuf = _scatter(d_s.reshape(T * TOPK), idx_init)
    xgs = [_gather_rows(idxbuf, x, EG * CAP, idx_off=g * EG * CAP)
           for g in range(NG)]
    outbuf = None
    for g in range(NG):
        outbuf = _experts_group(xgs[g], expert_gate_w, expert_up_w,
                                expert_down_w, outbuf, g)
    moe = _gather_rows(d_g.T.reshape(T * TOPK), outbuf, T * TOPK)

    final = _final(x, moe, w01,
                   shared_gate_w, shared_up_w, shared_down_w,
                   shared_expert_gate_w)
    return final.reshape(Bsz, Sl, Hd), logits


# speedup vs baseline: 1.0145x; 1.0145x over previous
import functools

import jax
import jax.numpy as jnp
from jax import lax
from jax.experimental import pallas as pl
from jax.experimental.pallas import tpu as pltpu
from jax.experimental.pallas import tpu_sc as plsc

E = 64
TOPK = 2
H = 768
I_EXP = 256
I_SH = 1536
T = 2048
CAP = 256
NSLOT = E * CAP
TRASH = NSLOT
IDXBUF = NSLOT + 16
DUMMY_ROW = T

NC, NS = 2, 16
NW = NC * NS


def _sigmoid(x):
    return 1.0 / (1.0 + jnp.exp(-x))


def _silu(x):
    return x * _sigmoid(x)


def _router_body(x_ref, gw_ref, logits_ref, ds_ref, dg_ref, w_ref, c_ref, oh_ref):
    x = x_ref[...]
    gw = gw_ref[...]
    logits = lax.dot_general(x, gw, (((1,), (1,)), ((), ())),
                             preferred_element_type=jnp.float32)
    logits_ref[...] = logits
    m = jnp.max(logits, axis=1, keepdims=True)
    ex = jnp.exp(logits - m)
    rw = ex / jnp.sum(ex, axis=1, keepdims=True)

    ii = lax.broadcasted_iota(jnp.int32, (T, E), 1)
    m1 = jnp.max(rw, axis=1, keepdims=True)
    a1 = jnp.min(jnp.where(rw == m1, ii, E), axis=1, keepdims=True)
    rw2 = jnp.where(ii == a1, -1.0, rw)
    m2 = jnp.max(rw2, axis=1, keepdims=True)
    a2 = jnp.min(jnp.where(rw2 == m2, ii, E), axis=1, keepdims=True)

    denom = jnp.maximum(m1 + m2, 1e-6)
    w0 = m1 / denom
    w1 = m2 / denom

    oh1 = (ii == a1).astype(jnp.float32)
    oh2 = (ii == a2).astype(jnp.float32)
    oh_ref[...] = oh1 + oh2

    r = lax.broadcasted_iota(jnp.int32, (256, 256), 0)
    c = lax.broadcasted_iota(jnp.int32, (256, 256), 1)
    tri = (c < r).astype(jnp.float32)

    def blk(i, carry):
        ohb = oh_ref[pl.ds(i * 256, 256), :]
        cb = lax.dot_general(tri, ohb, (((1,), (0,)), ((), ())),
                             preferred_element_type=jnp.float32)
        c_ref[pl.ds(i * 256, 256), :] = cb + carry
        return carry + jnp.sum(ohb, axis=0, keepdims=True)

    lax.fori_loop(0, 8, blk, jnp.zeros((1, E), jnp.float32))

    cexcl = c_ref[...]
    pos0 = jnp.sum(cexcl * oh1, axis=1, keepdims=True).astype(jnp.int32)
    pos1 = jnp.sum(cexcl * oh2, axis=1, keepdims=True).astype(jnp.int32)

    d0 = a1 * CAP + pos0
    d1 = a2 * CAP + pos1
    v0 = pos0 < CAP
    v1 = pos1 < CAP
    ds_ref[...] = jnp.concatenate(
        [jnp.where(v0, d0, TRASH), jnp.where(v1, d1, TRASH)], axis=1)
    dg_ref[...] = jnp.concatenate(
        [jnp.where(v0, d0, 0), jnp.where(v1, d1, 0)], axis=1)
    w_ref[...] = jnp.concatenate(
        [jnp.where(v0, w0, 0.0), jnp.where(v1, w1, 0.0)], axis=1)


def _router(x, gate_w):
    return pl.pallas_call(
        _router_body,
        out_shape=(
            jax.ShapeDtypeStruct((T, E), jnp.float32),
            jax.ShapeDtypeStruct((T, TOPK), jnp.int32),
            jax.ShapeDtypeStruct((T, TOPK), jnp.int32),
            jax.ShapeDtypeStruct((T, TOPK), jnp.float32),
        ),
        scratch_shapes=[pltpu.VMEM((T, E), jnp.float32),
                        pltpu.VMEM((T, E), jnp.float32)],
    )(x, gate_w)


def _scatter_body(ds_hbm, init_hbm, out_hbm, idxv, dstv, sem):
    ci = lax.axis_index("c")
    si = lax.axis_index("s")

    @pl.when(jnp.logical_and(ci == 0, si == 0))
    def _():
        pltpu.sync_copy(init_hbm, idxv)
        pltpu.sync_copy(ds_hbm, dstv)
        lanes = lax.iota(jnp.int32, 16)

        def step(j, _):
            idx = dstv[pl.ds(j * 16, 16)]
            tok = (j * 16 + lanes) >> 1
            plsc.store_scatter(idxv, [idx], tok)
            return 0

        lax.fori_loop(0, (T * TOPK) // 16, step, 0)
        pltpu.sync_copy(idxv, out_hbm)


def _scatter(ds_flat, idx_init):
    k = functools.partial(
        pl.kernel,
        out_type=jax.ShapeDtypeStruct((IDXBUF,), jnp.int32),
        mesh=plsc.VectorSubcoreMesh(core_axis_name="c", subcore_axis_name="s"),
        scratch_types=[
            pltpu.VMEM((IDXBUF,), jnp.int32),
            pltpu.VMEM((T * TOPK,), jnp.int32),
            pltpu.SemaphoreType.DMA,
        ],
        compiler_params=pltpu.CompilerParams(needs_layout_passes=False),
    )(_scatter_body)
    return k(ds_flat, idx_init)


def _gather_rows_body(idx_hbm, tab_hbm, out_hbm, idxv, rows, sem, *,
                      nrows, chunk, idx_off):
    ci = lax.axis_index("c")
    si = lax.axis_index("s")
    wid = si * NC + ci
    per_w = nrows // NW
    base = wid * per_w

    def step(ch, _):
        off = base + ch * chunk
        pltpu.sync_copy(idx_hbm.at[pl.ds(idx_off + off, chunk)], idxv)
        pltpu.async_copy(tab_hbm.at[idxv], rows, sem).wait()
        pltpu.sync_copy(rows, out_hbm.at[pl.ds(off, chunk)])
        return 0

    lax.fori_loop(0, per_w // chunk, step, 0)


def _gather_rows(idx, table, nrows, chunk=128, idx_off=0):
    body = functools.partial(_gather_rows_body, nrows=nrows, chunk=chunk,
                             idx_off=idx_off)
    k = functools.partial(
        pl.kernel,
        out_type=jax.ShapeDtypeStruct((nrows, H), jnp.float32),
        mesh=plsc.VectorSubcoreMesh(core_axis_name="c", subcore_axis_name="s"),
        scratch_types=[
            pltpu.VMEM((chunk,), jnp.int32),
            pltpu.VMEM((chunk, H), jnp.float32),
            pltpu.SemaphoreType.DMA,
        ],
    )(body)
    return k(idx, table)


NG = 4
EG = E // NG


def _experts_body(xg_ref, eg_ref, eu_ref, ed_ref, *rest):
    out_ref = rest[-1]
    cur = xg_ref[...]
    g = lax.dot_general(cur, eg_ref[0], (((1,), (1,)), ((), ())),
                        preferred_element_type=jnp.float32)
    u = lax.dot_general(cur, eu_ref[0], (((1,), (1,)), ((), ())),
                        preferred_element_type=jnp.float32)
    h = _silu(g) * u
    out_ref[...] = lax.dot_general(h, ed_ref[0], (((1,), (1,)), ((), ())),
                                   preferred_element_type=jnp.float32)


def _experts_group(xg_g, eg, eu, ed, prev, g):
    in_specs = [
        pl.BlockSpec((CAP, H), lambda e: (e, 0)),
        pl.BlockSpec((1, I_EXP, H), lambda e: (g * EG + e, 0, 0)),
        pl.BlockSpec((1, I_EXP, H), lambda e: (g * EG + e, 0, 0)),
        pl.BlockSpec((1, H, I_EXP), lambda e: (g * EG + e, 0, 0)),
    ]
    args = [xg_g, eg, eu, ed]
    aliases = {}
    if prev is not None:
        in_specs.append(pl.BlockSpec(memory_space=pltpu.MemorySpace.HBM))
        args.append(prev)
        aliases = {4: 0}
    return pl.pallas_call(
        _experts_body,
        grid=(EG,),
        in_specs=in_specs,
        out_specs=pl.BlockSpec((CAP, H), lambda e: (g * EG + e, 0)),
        out_shape=jax.ShapeDtypeStruct((NSLOT, H), jnp.float32),
        input_output_aliases=aliases,
    )(*args)


def _final_body(x_ref, m0_ref, m1_ref, w_ref, sg_ref, su_ref, sd_ref, seg_ref,
                out_ref, sgb_ref, sub_ref, sdb_ref):
    @pl.when(pl.program_id(0) == 0)
    def _():
        sgb_ref[...] = sg_ref[...].astype(jnp.bfloat16)
        sub_ref[...] = su_ref[...].astype(jnp.bfloat16)
        sdb_ref[...] = sd_ref[...].astype(jnp.bfloat16)

    xb = x_ref[...]
    xbb = xb.astype(jnp.bfloat16)
    wv = w_ref[...]
    moe_sum = m0_ref[...] * wv[:, 0:1] + m1_ref[...] * wv[:, 1:2]

    g = lax.dot_general(xbb, sgb_ref[...], (((1,), (1,)), ((), ())),
                        preferred_element_type=jnp.float32)
    u = lax.dot_general(xbb, sub_ref[...], (((1,), (1,)), ((), ())),
                        preferred_element_type=jnp.float32)
    h = (_silu(g) * u).astype(jnp.bfloat16)
    s = lax.dot_general(h, sdb_ref[...], (((1,), (1,)), ((), ())),
                        preferred_element_type=jnp.float32)
    gate = _sigmoid(lax.dot_general(xb, seg_ref[...], (((1,), (1,)), ((), ())),
                                    preferred_element_type=jnp.float32))
    out_ref[...] = moe_sum + gate * s


def _final(x, moe, w01, sgw, suw, sdw, segw):
    nblk = T // 256
    return pl.pallas_call(
        _final_body,
        grid=(nblk,),
        in_specs=[
            pl.BlockSpec((256, H), lambda i: (i, 0)),
            pl.BlockSpec((256, H), lambda i: (i, 0)),
            pl.BlockSpec((256, H), lambda i: (i + nblk, 0)),
            pl.BlockSpec((256, TOPK), lambda i: (i, 0)),
            pl.BlockSpec((I_SH, H), lambda i: (0, 0)),
            pl.BlockSpec((I_SH, H), lambda i: (0, 0)),
            pl.BlockSpec((H, I_SH), lambda i: (0, 0)),
            pl.BlockSpec((1, H), lambda i: (0, 0)),
        ],
        out_specs=pl.BlockSpec((256, H), lambda i: (i, 0)),
        out_shape=jax.ShapeDtypeStruct((T, H), jnp.float32),
        scratch_shapes=[
            pltpu.VMEM((I_SH, H), jnp.bfloat16),
            pltpu.VMEM((I_SH, H), jnp.bfloat16),
            pltpu.VMEM((H, I_SH), jnp.bfloat16),
        ],
    )(x, moe, moe, w01, sgw, suw, sdw, segw)


def kernel(hidden_states, gate_w, expert_gate_w, expert_up_w, expert_down_w,
           shared_gate_w, shared_up_w, shared_down_w, shared_expert_gate_w):
    Bsz, Sl, Hd = hidden_states.shape
    x = hidden_states.reshape(T, H)

    logits, d_s, d_g, w01 = _router(x, gate_w)

    idx_init = lax.iota(jnp.int32, IDXBUF) % T
    idxbuf = _scatter(d_s.reshape(T * TOPK), idx_init)
    xgs = [_gather_rows(idxbuf, x, EG * CAP, idx_off=g * EG * CAP)
           for g in range(NG)]
    outbuf = None
    for g in range(NG):
        outbuf = _experts_group(xgs[g], expert_gate_w, expert_up_w,
                                expert_down_w, outbuf, g)
    moe = _gather_rows(d_g.T.reshape(T * TOPK), outbuf, T * TOPK)

    final = _final(x, moe, w01,
                   shared_gate_w, shared_up_w, shared_down_w,
                   shared_expert_gate_w)
    return final.reshape(Bsz, Sl, Hd), logits


# speedup vs baseline: 1.0215x; 1.0069x over previous
import functools

import jax
import jax.numpy as jnp
from jax import lax
from jax.experimental import pallas as pl
from jax.experimental.pallas import tpu as pltpu
from jax.experimental.pallas import tpu_sc as plsc

E = 64
TOPK = 2
H = 768
I_EXP = 256
I_SH = 1536
T = 2048
CAP = 256
NSLOT = E * CAP
TRASH = NSLOT
IDXBUF = NSLOT + 16
DUMMY_ROW = T

NC, NS = 2, 16
NW = NC * NS


def _sigmoid(x):
    return 1.0 / (1.0 + jnp.exp(-x))


def _silu(x):
    return x * _sigmoid(x)


def _router_body(x_ref, gw_ref, logits_ref, ds_ref, dg_ref, w_ref, c_ref, oh_ref):
    x = x_ref[...]
    gw = gw_ref[...]
    logits = lax.dot_general(x, gw, (((1,), (1,)), ((), ())),
                             preferred_element_type=jnp.float32)
    logits_ref[...] = logits
    m = jnp.max(logits, axis=1, keepdims=True)
    ex = jnp.exp(logits - m)
    rw = ex / jnp.sum(ex, axis=1, keepdims=True)

    ii = lax.broadcasted_iota(jnp.int32, (T, E), 1)
    m1 = jnp.max(rw, axis=1, keepdims=True)
    a1 = jnp.min(jnp.where(rw == m1, ii, E), axis=1, keepdims=True)
    rw2 = jnp.where(ii == a1, -1.0, rw)
    m2 = jnp.max(rw2, axis=1, keepdims=True)
    a2 = jnp.min(jnp.where(rw2 == m2, ii, E), axis=1, keepdims=True)

    denom = jnp.maximum(m1 + m2, 1e-6)
    w0 = m1 / denom
    w1 = m2 / denom

    oh1 = (ii == a1).astype(jnp.float32)
    oh2 = (ii == a2).astype(jnp.float32)
    oh_ref[...] = oh1 + oh2

    r = lax.broadcasted_iota(jnp.int32, (256, 256), 0)
    c = lax.broadcasted_iota(jnp.int32, (256, 256), 1)
    tri = (c < r).astype(jnp.float32)

    def blk(i, carry):
        ohb = oh_ref[pl.ds(i * 256, 256), :]
        cb = lax.dot_general(tri, ohb, (((1,), (0,)), ((), ())),
                             preferred_element_type=jnp.float32)
        c_ref[pl.ds(i * 256, 256), :] = cb + carry
        return carry + jnp.sum(ohb, axis=0, keepdims=True)

    lax.fori_loop(0, 8, blk, jnp.zeros((1, E), jnp.float32))

    cexcl = c_ref[...]
    pos0 = jnp.sum(cexcl * oh1, axis=1, keepdims=True).astype(jnp.int32)
    pos1 = jnp.sum(cexcl * oh2, axis=1, keepdims=True).astype(jnp.int32)

    d0 = a1 * CAP + pos0
    d1 = a2 * CAP + pos1
    v0 = pos0 < CAP
    v1 = pos1 < CAP
    ds_ref[...] = jnp.concatenate(
        [jnp.where(v0, d0, TRASH), jnp.where(v1, d1, TRASH)], axis=1)
    dg_ref[...] = jnp.concatenate(
        [jnp.where(v0, d0, 0), jnp.where(v1, d1, 0)], axis=1)
    w_ref[...] = jnp.concatenate(
        [jnp.where(v0, w0, 0.0), jnp.where(v1, w1, 0.0)], axis=1)


def _router(x, gate_w):
    return pl.pallas_call(
        _router_body,
        out_shape=(
            jax.ShapeDtypeStruct((T, E), jnp.float32),
            jax.ShapeDtypeStruct((T, TOPK), jnp.int32),
            jax.ShapeDtypeStruct((T, TOPK), jnp.int32),
            jax.ShapeDtypeStruct((T, TOPK), jnp.float32),
        ),
        scratch_shapes=[pltpu.VMEM((T, E), jnp.float32),
                        pltpu.VMEM((T, E), jnp.float32)],
    )(x, gate_w)


def _scatter_body(ds_hbm, init_hbm, out_hbm, idxv, dstv, sem):
    ci = lax.axis_index("c")
    si = lax.axis_index("s")

    @pl.when(jnp.logical_and(ci == 0, si == 0))
    def _():
        pltpu.sync_copy(init_hbm, idxv)
        pltpu.sync_copy(ds_hbm, dstv)
        lanes = lax.iota(jnp.int32, 16)

        def step(j, _):
            idx = dstv[pl.ds(j * 16, 16)]
            tok = (j * 16 + lanes) >> 1
            plsc.store_scatter(idxv, [idx], tok)
            return 0

        lax.fori_loop(0, (T * TOPK) // 16, step, 0)
        pltpu.sync_copy(idxv, out_hbm)


def _scatter(ds_flat, idx_init):
    k = functools.partial(
        pl.kernel,
        out_type=jax.ShapeDtypeStruct((IDXBUF,), jnp.int32),
        mesh=plsc.VectorSubcoreMesh(core_axis_name="c", subcore_axis_name="s"),
        scratch_types=[
            pltpu.VMEM((IDXBUF,), jnp.int32),
            pltpu.VMEM((T * TOPK,), jnp.int32),
            pltpu.SemaphoreType.DMA,
        ],
        compiler_params=pltpu.CompilerParams(needs_layout_passes=False),
    )(_scatter_body)
    return k(ds_flat, idx_init)


def _gather_rows_body(idx_hbm, tab_hbm, out_hbm, idxv, rows, sem, *,
                      nrows, chunk, idx_off):
    ci = lax.axis_index("c")
    si = lax.axis_index("s")
    wid = si * NC + ci
    per_w = nrows // NW
    base = wid * per_w

    def step(ch, _):
        off = base + ch * chunk
        pltpu.sync_copy(idx_hbm.at[pl.ds(idx_off + off, chunk)], idxv)
        pltpu.async_copy(tab_hbm.at[idxv], rows, sem).wait()
        pltpu.sync_copy(rows, out_hbm.at[pl.ds(off, chunk)])
        return 0

    lax.fori_loop(0, per_w // chunk, step, 0)


def _gather_rows(idx, table, nrows, chunk=128, idx_off=0):
    body = functools.partial(_gather_rows_body, nrows=nrows, chunk=chunk,
                             idx_off=idx_off)
    k = functools.partial(
        pl.kernel,
        out_type=jax.ShapeDtypeStruct((nrows, H), jnp.float32),
        mesh=plsc.VectorSubcoreMesh(core_axis_name="c", subcore_axis_name="s"),
        scratch_types=[
            pltpu.VMEM((chunk,), jnp.int32),
            pltpu.VMEM((chunk, H), jnp.float32),
            pltpu.SemaphoreType.DMA,
        ],
    )(body)
    return k(idx, table)


GROUPS = (8, 16, 20, 20)
GSTART = (0, 8, 24, 44)


def _experts_body(xg_ref, eg_ref, eu_ref, ed_ref, *rest):
    out_ref = rest[-1]
    cur = xg_ref[...]
    g = lax.dot_general(cur, eg_ref[0], (((1,), (1,)), ((), ())),
                        preferred_element_type=jnp.float32)
    u = lax.dot_general(cur, eu_ref[0], (((1,), (1,)), ((), ())),
                        preferred_element_type=jnp.float32)
    h = _silu(g) * u
    out_ref[...] = lax.dot_general(h, ed_ref[0], (((1,), (1,)), ((), ())),
                                   preferred_element_type=jnp.float32)


def _experts_group(xg_g, eg, eu, ed, prev, g):
    g0 = GSTART[g]
    in_specs = [
        pl.BlockSpec((CAP, H), lambda e: (e, 0)),
        pl.BlockSpec((1, I_EXP, H), lambda e: (g0 + e, 0, 0)),
        pl.BlockSpec((1, I_EXP, H), lambda e: (g0 + e, 0, 0)),
        pl.BlockSpec((1, H, I_EXP), lambda e: (g0 + e, 0, 0)),
    ]
    args = [xg_g, eg, eu, ed]
    aliases = {}
    if prev is not None:
        in_specs.append(pl.BlockSpec(memory_space=pltpu.MemorySpace.HBM))
        args.append(prev)
        aliases = {4: 0}
    return pl.pallas_call(
        _experts_body,
        grid=(GROUPS[g],),
        in_specs=in_specs,
        out_specs=pl.BlockSpec((CAP, H), lambda e: (g0 + e, 0)),
        out_shape=jax.ShapeDtypeStruct((NSLOT, H), jnp.float32),
        input_output_aliases=aliases,
    )(*args)


def _final_body(x_ref, m0_ref, m1_ref, w_ref, sg_ref, su_ref, sd_ref, seg_ref,
                out_ref):
    xb = x_ref[...]
    wv = w_ref[...]
    moe_sum = m0_ref[...] * wv[:, 0:1] + m1_ref[...] * wv[:, 1:2]

    g = lax.dot_general(xb, sg_ref[...], (((1,), (1,)), ((), ())),
                        preferred_element_type=jnp.float32)
    u = lax.dot_general(xb, su_ref[...], (((1,), (1,)), ((), ())),
                        preferred_element_type=jnp.float32)
    s = lax.dot_general(_silu(g) * u, sd_ref[...], (((1,), (1,)), ((), ())),
                        preferred_element_type=jnp.float32)
    gate = _sigmoid(lax.dot_general(xb, seg_ref[...], (((1,), (1,)), ((), ())),
                                    preferred_element_type=jnp.float32))
    out_ref[...] = moe_sum + gate * s


def _final(x, moe, w01, sgw, suw, sdw, segw):
    nblk = T // 256
    return pl.pallas_call(
        _final_body,
        grid=(nblk,),
        in_specs=[
            pl.BlockSpec((256, H), lambda i: (i, 0)),
            pl.BlockSpec((256, H), lambda i: (i, 0)),
            pl.BlockSpec((256, H), lambda i: (i + nblk, 0)),
            pl.BlockSpec((256, TOPK), lambda i: (i, 0)),
            pl.BlockSpec((I_SH, H), lambda i: (0, 0)),
            pl.BlockSpec((I_SH, H), lambda i: (0, 0)),
            pl.BlockSpec((H, I_SH), lambda i: (0, 0)),
            pl.BlockSpec((1, H), lambda i: (0, 0)),
        ],
        out_specs=pl.BlockSpec((256, H), lambda i: (i, 0)),
        out_shape=jax.ShapeDtypeStruct((T, H), jnp.float32),
    )(x, moe, moe, w01, sgw, suw, sdw, segw)


def kernel(hidden_states, gate_w, expert_gate_w, expert_up_w, expert_down_w,
           shared_gate_w, shared_up_w, shared_down_w, shared_expert_gate_w):
    Bsz, Sl, Hd = hidden_states.shape
    x = hidden_states.reshape(T, H)

    logits, d_s, d_g, w01 = _router(x, gate_w)

    idx_init = lax.iota(jnp.int32, IDXBUF) % T
    idxbuf = _scatter(d_s.reshape(T * TOPK), idx_init)
    outbuf = None
    xgs = []
    for g in range(len(GROUPS)):
        nrows = GROUPS[g] * CAP
        per_w = nrows // NW
        chunk = per_w if per_w <= 128 else per_w // ((per_w + 127) // 128)
        xgs.append(_gather_rows(idxbuf, x, nrows, chunk=chunk,
                                idx_off=GSTART[g] * CAP))
    for g in range(len(GROUPS)):
        outbuf = _experts_group(xgs[g], expert_gate_w, expert_up_w,
                                expert_down_w, outbuf, g)
    moe = _gather_rows(d_g.T.reshape(T * TOPK), outbuf, T * TOPK)

    final = _final(x, moe, w01,
                   shared_gate_w, shared_up_w, shared_down_w,
                   shared_expert_gate_w)
    return final.reshape(Bsz, Sl, Hd), logits


# speedup vs baseline: 1.1071x; 1.0838x over previous
import functools

import jax
import jax.numpy as jnp
from jax import lax
from jax.experimental import pallas as pl
from jax.experimental.pallas import tpu as pltpu
from jax.experimental.pallas import tpu_sc as plsc

E = 64
TOPK = 2
H = 768
I_EXP = 256
I_SH = 1536
T = 2048
CAP = 256
NSLOT = E * CAP
TRASH = NSLOT
IDXBUF = NSLOT + 16
DUMMY_ROW = T

NC, NS = 2, 16
NW = NC * NS


def _sigmoid(x):
    return 1.0 / (1.0 + jnp.exp(-x))


def _silu(x):
    return x * _sigmoid(x)


def _router_body(x_ref, gw_ref, logits_ref, ds_ref, dg_ref, w_ref, c_ref, oh_ref):
    x = x_ref[...]
    gw = gw_ref[...]
    logits = lax.dot_general(x, gw, (((1,), (1,)), ((), ())),
                             preferred_element_type=jnp.float32)
    logits_ref[...] = logits
    m = jnp.max(logits, axis=1, keepdims=True)
    ex = jnp.exp(logits - m)
    rw = ex / jnp.sum(ex, axis=1, keepdims=True)

    ii = lax.broadcasted_iota(jnp.int32, (T, E), 1)
    m1 = jnp.max(rw, axis=1, keepdims=True)
    a1 = jnp.min(jnp.where(rw == m1, ii, E), axis=1, keepdims=True)
    rw2 = jnp.where(ii == a1, -1.0, rw)
    m2 = jnp.max(rw2, axis=1, keepdims=True)
    a2 = jnp.min(jnp.where(rw2 == m2, ii, E), axis=1, keepdims=True)

    denom = jnp.maximum(m1 + m2, 1e-6)
    w0 = m1 / denom
    w1 = m2 / denom

    oh1 = (ii == a1).astype(jnp.float32)
    oh2 = (ii == a2).astype(jnp.float32)
    oh_ref[...] = oh1 + oh2

    r = lax.broadcasted_iota(jnp.int32, (256, 256), 0)
    c = lax.broadcasted_iota(jnp.int32, (256, 256), 1)
    tri = (c < r).astype(jnp.float32)

    def blk(i, carry):
        ohb = oh_ref[pl.ds(i * 256, 256), :]
        cb = lax.dot_general(tri, ohb, (((1,), (0,)), ((), ())),
                             preferred_element_type=jnp.float32)
        c_ref[pl.ds(i * 256, 256), :] = cb + carry
        return carry + jnp.sum(ohb, axis=0, keepdims=True)

    lax.fori_loop(0, 8, blk, jnp.zeros((1, E), jnp.float32))

    cexcl = c_ref[...]
    pos0 = jnp.sum(cexcl * oh1, axis=1, keepdims=True).astype(jnp.int32)
    pos1 = jnp.sum(cexcl * oh2, axis=1, keepdims=True).astype(jnp.int32)

    d0 = a1 * CAP + pos0
    d1 = a2 * CAP + pos1
    v0 = pos0 < CAP
    v1 = pos1 < CAP
    ds_ref[...] = jnp.concatenate(
        [jnp.where(v0, d0, TRASH), jnp.where(v1, d1, TRASH)], axis=1)
    dg_ref[...] = jnp.concatenate(
        [jnp.where(v0, d0, 0), jnp.where(v1, d1, 0)], axis=1)
    w_ref[...] = jnp.concatenate(
        [jnp.where(v0, w0, 0.0), jnp.where(v1, w1, 0.0)], axis=1)


def _router(x, gate_w):
    return pl.pallas_call(
        _router_body,
        out_shape=(
            jax.ShapeDtypeStruct((T, E), jnp.float32),
            jax.ShapeDtypeStruct((T, TOPK), jnp.int32),
            jax.ShapeDtypeStruct((T, TOPK), jnp.int32),
            jax.ShapeDtypeStruct((T, TOPK), jnp.float32),
        ),
        scratch_shapes=[pltpu.VMEM((T, E), jnp.float32),
                        pltpu.VMEM((T, E), jnp.float32)],
    )(x, gate_w)


def _scatter_body(ds_hbm, init_hbm, out_hbm, idxv, dstv, sem):
    ci = lax.axis_index("c")
    si = lax.axis_index("s")

    @pl.when(jnp.logical_and(ci == 0, si == 0))
    def _():
        pltpu.sync_copy(init_hbm, idxv)
        pltpu.sync_copy(ds_hbm, dstv)
        lanes = lax.iota(jnp.int32, 16)

        def step(j, _):
            idx = dstv[pl.ds(j * 16, 16)]
            tok = (j * 16 + lanes) >> 1
            plsc.store_scatter(idxv, [idx], tok)
            return 0

        lax.fori_loop(0, (T * TOPK) // 16, step, 0)
        pltpu.sync_copy(idxv, out_hbm)


def _scatter(ds_flat, idx_init):
    k = functools.partial(
        pl.kernel,
        out_type=jax.ShapeDtypeStruct((IDXBUF,), jnp.int32),
        mesh=plsc.VectorSubcoreMesh(core_axis_name="c", subcore_axis_name="s"),
        scratch_types=[
            pltpu.VMEM((IDXBUF,), jnp.int32),
            pltpu.VMEM((T * TOPK,), jnp.int32),
            pltpu.SemaphoreType.DMA,
        ],
        compiler_params=pltpu.CompilerParams(needs_layout_passes=False),
    )(_scatter_body)
    return k(ds_flat, idx_init)


def _gather_rows_body(idx_hbm, tab_hbm, out_hbm, idxv, rows, sem, *,
                      nrows, chunk, idx_off, dtype=jnp.float32):
    ci = lax.axis_index("c")
    si = lax.axis_index("s")
    wid = si * NC + ci
    per_w = nrows // NW
    base = wid * per_w

    def step(ch, _):
        off = base + ch * chunk
        pltpu.sync_copy(idx_hbm.at[pl.ds(idx_off + off, chunk)], idxv)
        pltpu.async_copy(tab_hbm.at[idxv], rows, sem).wait()
        pltpu.sync_copy(rows, out_hbm.at[pl.ds(off, chunk)])
        return 0

    lax.fori_loop(0, per_w // chunk, step, 0)


def _gather_rows(idx, table, nrows, chunk=128, idx_off=0):
    dtype = table.dtype
    width = table.shape[1]
    body = functools.partial(_gather_rows_body, nrows=nrows, chunk=chunk,
                             idx_off=idx_off)
    k = functools.partial(
        pl.kernel,
        out_type=jax.ShapeDtypeStruct((nrows, width), dtype),
        mesh=plsc.VectorSubcoreMesh(core_axis_name="c", subcore_axis_name="s"),
        scratch_types=[
            pltpu.VMEM((chunk,), jnp.int32),
            pltpu.VMEM((chunk, width), dtype),
            pltpu.SemaphoreType.DMA,
        ],
    )(body)
    return k(idx, table)


GROUPS = (8, 16, 20, 20)
GSTART = (0, 8, 24, 44)


def _experts_body(xg_ref, eg_ref, eu_ref, ed_ref, *rest):
    out_ref = rest[-1]
    v = xg_ref[...]
    lo = lax.bitcast_convert_type(v << 16, jnp.float32)
    hi = lax.bitcast_convert_type(v & jnp.int32(-65536), jnp.float32)
    cur = jnp.concatenate([lo, hi], axis=1)
    g = lax.dot_general(cur, eg_ref[0], (((1,), (1,)), ((), ())),
                        preferred_element_type=jnp.float32)
    u = lax.dot_general(cur, eu_ref[0], (((1,), (1,)), ((), ())),
                        preferred_element_type=jnp.float32)
    h = _silu(g) * u
    out_ref[...] = lax.dot_general(h, ed_ref[0], (((1,), (1,)), ((), ())),
                                   preferred_element_type=jnp.float32)


def _experts_group(xg_g, eg, eu, ed, prev, g):
    g0 = GSTART[g]
    in_specs = [
        pl.BlockSpec((CAP, H // 2), lambda e: (e, 0)),
        pl.BlockSpec((1, I_EXP, H), lambda e: (g0 + e, 0, 0)),
        pl.BlockSpec((1, I_EXP, H), lambda e: (g0 + e, 0, 0)),
        pl.BlockSpec((1, H, I_EXP), lambda e: (g0 + e, 0, 0)),
    ]
    args = [xg_g, eg, eu, ed]
    aliases = {}
    if prev is not None:
        in_specs.append(pl.BlockSpec(memory_space=pltpu.MemorySpace.HBM))
        args.append(prev)
        aliases = {4: 0}
    return pl.pallas_call(
        _experts_body,
        grid=(GROUPS[g],),
        in_specs=in_specs,
        out_specs=pl.BlockSpec((CAP, H), lambda e: (g0 + e, 0)),
        out_shape=jax.ShapeDtypeStruct((NSLOT, H), jnp.float32),
        input_output_aliases=aliases,
    )(*args)


def _final_body(x_ref, m0_ref, m1_ref, w_ref, sg_ref, su_ref, sd_ref, seg_ref,
                out_ref):
    xb = x_ref[...]
    wv = w_ref[...]
    moe_sum = m0_ref[...] * wv[:, 0:1] + m1_ref[...] * wv[:, 1:2]

    g = lax.dot_general(xb, sg_ref[...], (((1,), (1,)), ((), ())),
                        preferred_element_type=jnp.float32)
    u = lax.dot_general(xb, su_ref[...], (((1,), (1,)), ((), ())),
                        preferred_element_type=jnp.float32)
    s = lax.dot_general(_silu(g) * u, sd_ref[...], (((1,), (1,)), ((), ())),
                        preferred_element_type=jnp.float32)
    gate = _sigmoid(lax.dot_general(xb, seg_ref[...], (((1,), (1,)), ((), ())),
                                    preferred_element_type=jnp.float32))
    out_ref[...] = moe_sum + gate * s


def _final(x, moe, w01, sgw, suw, sdw, segw):
    nblk = T // 256
    return pl.pallas_call(
        _final_body,
        grid=(nblk,),
        in_specs=[
            pl.BlockSpec((256, H), lambda i: (i, 0)),
            pl.BlockSpec((256, H), lambda i: (i, 0)),
            pl.BlockSpec((256, H), lambda i: (i + nblk, 0)),
            pl.BlockSpec((256, TOPK), lambda i: (i, 0)),
            pl.BlockSpec((I_SH, H), lambda i: (0, 0)),
            pl.BlockSpec((I_SH, H), lambda i: (0, 0)),
            pl.BlockSpec((H, I_SH), lambda i: (0, 0)),
            pl.BlockSpec((1, H), lambda i: (0, 0)),
        ],
        out_specs=pl.BlockSpec((256, H), lambda i: (i, 0)),
        out_shape=jax.ShapeDtypeStruct((T, H), jnp.float32),
    )(x, moe, moe, w01, sgw, suw, sdw, segw)


def kernel(hidden_states, gate_w, expert_gate_w, expert_up_w, expert_down_w,
           shared_gate_w, shared_up_w, shared_down_w, shared_expert_gate_w):
    Bsz, Sl, Hd = hidden_states.shape
    x = hidden_states.reshape(T, H)

    logits, d_s, d_g, w01 = _router(x, gate_w)

    idx_init = lax.iota(jnp.int32, IDXBUF) % T
    idxbuf = _scatter(d_s.reshape(T * TOPK), idx_init)
    outbuf = None
    xgs = []
    xb16 = x.astype(jnp.bfloat16)
    x16 = lax.bitcast_convert_type(
        jnp.stack([xb16[:, :H // 2], xb16[:, H // 2:]], axis=-1), jnp.int32)
    for g in range(len(GROUPS)):
        nrows = GROUPS[g] * CAP
        per_w = nrows // NW
        chunk = per_w if per_w <= 128 else per_w // ((per_w + 127) // 128)
        xgs.append(_gather_rows(idxbuf, x16, nrows, chunk=chunk,
                                idx_off=GSTART[g] * CAP))
    for g in range(len(GROUPS)):
        outbuf = _experts_group(xgs[g], expert_gate_w, expert_up_w,
                                expert_down_w, outbuf, g)
    moe = _gather_rows(d_g.T.reshape(T * TOPK), outbuf, T * TOPK)

    final = _final(x, moe, w01,
                   shared_gate_w, shared_up_w, shared_down_w,
                   shared_expert_gate_w)
    return final.reshape(Bsz, Sl, Hd), logits


# speedup vs baseline: 1.1734x; 1.0599x over previous
import functools

import jax
import jax.numpy as jnp
from jax import lax
from jax.experimental import pallas as pl
from jax.experimental.pallas import tpu as pltpu
from jax.experimental.pallas import tpu_sc as plsc

E = 64
TOPK = 2
H = 768
I_EXP = 256
I_SH = 1536
T = 2048
CAP = 256
NSLOT = E * CAP
TRASH = NSLOT
IDXBUF = NSLOT + 16
DUMMY_ROW = T

NC, NS = 2, 16
NW = NC * NS


def _sigmoid(x):
    return 1.0 / (1.0 + jnp.exp(-x))


def _silu(x):
    return x * _sigmoid(x)


def _router_body(x_ref, gw_ref, logits_ref, ds_ref, dg_ref, w_ref, c_ref, oh_ref):
    x = x_ref[...]
    gw = gw_ref[...]
    logits = lax.dot_general(x, gw, (((1,), (1,)), ((), ())),
                             preferred_element_type=jnp.float32)
    logits_ref[...] = logits
    m = jnp.max(logits, axis=1, keepdims=True)
    ex = jnp.exp(logits - m)
    rw = ex / jnp.sum(ex, axis=1, keepdims=True)

    ii = lax.broadcasted_iota(jnp.int32, (T, E), 1)
    m1 = jnp.max(rw, axis=1, keepdims=True)
    a1 = jnp.min(jnp.where(rw == m1, ii, E), axis=1, keepdims=True)
    rw2 = jnp.where(ii == a1, -1.0, rw)
    m2 = jnp.max(rw2, axis=1, keepdims=True)
    a2 = jnp.min(jnp.where(rw2 == m2, ii, E), axis=1, keepdims=True)

    denom = jnp.maximum(m1 + m2, 1e-6)
    w0 = m1 / denom
    w1 = m2 / denom

    oh1 = (ii == a1).astype(jnp.float32)
    oh2 = (ii == a2).astype(jnp.float32)
    oh_ref[...] = oh1 + oh2

    r = lax.broadcasted_iota(jnp.int32, (256, 256), 0)
    c = lax.broadcasted_iota(jnp.int32, (256, 256), 1)
    tri = (c < r).astype(jnp.float32)

    def blk(i, carry):
        ohb = oh_ref[pl.ds(i * 256, 256), :]
        cb = lax.dot_general(tri, ohb, (((1,), (0,)), ((), ())),
                             preferred_element_type=jnp.float32)
        c_ref[pl.ds(i * 256, 256), :] = cb + carry
        return carry + jnp.sum(ohb, axis=0, keepdims=True)

    lax.fori_loop(0, 8, blk, jnp.zeros((1, E), jnp.float32))

    cexcl = c_ref[...]
    pos0 = jnp.sum(cexcl * oh1, axis=1, keepdims=True).astype(jnp.int32)
    pos1 = jnp.sum(cexcl * oh2, axis=1, keepdims=True).astype(jnp.int32)

    d0 = a1 * CAP + pos0
    d1 = a2 * CAP + pos1
    v0 = pos0 < CAP
    v1 = pos1 < CAP
    ds_ref[...] = jnp.concatenate(
        [jnp.where(v0, d0, TRASH), jnp.where(v1, d1, TRASH)], axis=1)
    dg_ref[...] = jnp.concatenate(
        [jnp.where(v0, d0, 0), jnp.where(v1, d1, 0)], axis=1)
    w_ref[...] = jnp.concatenate(
        [jnp.where(v0, w0, 0.0), jnp.where(v1, w1, 0.0)], axis=1)


def _router(x, gate_w):
    return pl.pallas_call(
        _router_body,
        out_shape=(
            jax.ShapeDtypeStruct((T, E), jnp.float32),
            jax.ShapeDtypeStruct((T, TOPK), jnp.int32),
            jax.ShapeDtypeStruct((T, TOPK), jnp.int32),
            jax.ShapeDtypeStruct((T, TOPK), jnp.float32),
        ),
        scratch_shapes=[pltpu.VMEM((T, E), jnp.float32),
                        pltpu.VMEM((T, E), jnp.float32)],
    )(x, gate_w)


def _scatter_body(ds_hbm, init_hbm, out_hbm, idxv, dstv, sem):
    ci = lax.axis_index("c")
    si = lax.axis_index("s")

    @pl.when(jnp.logical_and(ci == 0, si == 0))
    def _():
        pltpu.sync_copy(init_hbm, idxv)
        pltpu.sync_copy(ds_hbm, dstv)
        lanes = lax.iota(jnp.int32, 16)

        def step(j, _):
            idx = dstv[pl.ds(j * 16, 16)]
            tok = (j * 16 + lanes) >> 1
            plsc.store_scatter(idxv, [idx], tok)
            return 0

        lax.fori_loop(0, (T * TOPK) // 16, step, 0)
        pltpu.sync_copy(idxv, out_hbm)


def _scatter(ds_flat, idx_init):
    k = functools.partial(
        pl.kernel,
        out_type=jax.ShapeDtypeStruct((IDXBUF,), jnp.int32),
        mesh=plsc.VectorSubcoreMesh(core_axis_name="c", subcore_axis_name="s"),
        scratch_types=[
            pltpu.VMEM((IDXBUF,), jnp.int32),
            pltpu.VMEM((T * TOPK,), jnp.int32),
            pltpu.SemaphoreType.DMA,
        ],
        compiler_params=pltpu.CompilerParams(needs_layout_passes=False),
    )(_scatter_body)
    return k(ds_flat, idx_init)


def _gather_rows_body(idx_hbm, tab_hbm, out_hbm, idxv, rows, sem, *,
                      nrows, chunk, idx_off, dtype=jnp.float32):
    ci = lax.axis_index("c")
    si = lax.axis_index("s")
    wid = si * NC + ci
    per_w = nrows // NW
    base = wid * per_w

    def step(ch, _):
        off = base + ch * chunk
        pltpu.sync_copy(idx_hbm.at[pl.ds(idx_off + off, chunk)], idxv)
        pltpu.async_copy(tab_hbm.at[idxv], rows, sem).wait()
        pltpu.sync_copy(rows, out_hbm.at[pl.ds(off, chunk)])
        return 0

    lax.fori_loop(0, per_w // chunk, step, 0)


def _gather_rows(idx, table, nrows, chunk=128, idx_off=0):
    dtype = table.dtype
    width = table.shape[1]
    body = functools.partial(_gather_rows_body, nrows=nrows, chunk=chunk,
                             idx_off=idx_off)
    k = functools.partial(
        pl.kernel,
        out_type=jax.ShapeDtypeStruct((nrows, width), dtype),
        mesh=plsc.VectorSubcoreMesh(core_axis_name="c", subcore_axis_name="s"),
        scratch_types=[
            pltpu.VMEM((chunk,), jnp.int32),
            pltpu.VMEM((chunk, width), dtype),
            pltpu.SemaphoreType.DMA,
        ],
    )(body)
    return k(idx, table)


GROUPS = (8, 16, 20, 20)
GSTART = (0, 8, 24, 44)


def _experts_body(xg_ref, eg_ref, eu_ref, ed_ref, *rest):
    out_ref = rest[-1]
    v = xg_ref[...]
    lo = lax.bitcast_convert_type(v << 16, jnp.float32)
    hi = lax.bitcast_convert_type(v & jnp.int32(-65536), jnp.float32)
    cur = jnp.concatenate([lo, hi], axis=1)
    g = lax.dot_general(cur, eg_ref[0], (((1,), (1,)), ((), ())),
                        preferred_element_type=jnp.float32)
    u = lax.dot_general(cur, eu_ref[0], (((1,), (1,)), ((), ())),
                        preferred_element_type=jnp.float32)
    h = _silu(g) * u
    out = lax.dot_general(h, ed_ref[0], (((1,), (1,)), ((), ())),
                          preferred_element_type=jnp.float32)
    bits = lax.bitcast_convert_type(out, jnp.int32) + jnp.int32(0x8000)
    lo = (bits[:, :H // 2] >> 16) & jnp.int32(0xFFFF)
    hi = bits[:, H // 2:] & jnp.int32(-65536)
    out_ref[...] = lo | hi


def _experts_group(xg_g, eg, eu, ed, prev, g):
    g0 = GSTART[g]
    in_specs = [
        pl.BlockSpec((CAP, H // 2), lambda e: (e, 0)),
        pl.BlockSpec((1, I_EXP, H), lambda e: (g0 + e, 0, 0)),
        pl.BlockSpec((1, I_EXP, H), lambda e: (g0 + e, 0, 0)),
        pl.BlockSpec((1, H, I_EXP), lambda e: (g0 + e, 0, 0)),
    ]
    args = [xg_g, eg, eu, ed]
    aliases = {}
    if prev is not None:
        in_specs.append(pl.BlockSpec(memory_space=pltpu.MemorySpace.HBM))
        args.append(prev)
        aliases = {4: 0}
    return pl.pallas_call(
        _experts_body,
        grid=(GROUPS[g],),
        in_specs=in_specs,
        out_specs=pl.BlockSpec((CAP, H // 2), lambda e: (g0 + e, 0)),
        out_shape=jax.ShapeDtypeStruct((NSLOT, H // 2), jnp.int32),
        input_output_aliases=aliases,
    )(*args)


def _final_body(x_ref, m0_ref, m1_ref, w_ref, sg_ref, su_ref, sd_ref, seg_ref,
                out_ref):
    xb = x_ref[...]
    wv = w_ref[...]

    def unpack(v):
        lo = lax.bitcast_convert_type(v << 16, jnp.float32)
        hi = lax.bitcast_convert_type(v & jnp.int32(-65536), jnp.float32)
        return jnp.concatenate([lo, hi], axis=1)

    moe_sum = (unpack(m0_ref[...]) * wv[:, 0:1] +
               unpack(m1_ref[...]) * wv[:, 1:2])

    g = lax.dot_general(xb, sg_ref[...], (((1,), (1,)), ((), ())),
                        preferred_element_type=jnp.float32)
    u = lax.dot_general(xb, su_ref[...], (((1,), (1,)), ((), ())),
                        preferred_element_type=jnp.float32)
    s = lax.dot_general(_silu(g) * u, sd_ref[...], (((1,), (1,)), ((), ())),
                        preferred_element_type=jnp.float32)
    gate = _sigmoid(lax.dot_general(xb, seg_ref[...], (((1,), (1,)), ((), ())),
                                    preferred_element_type=jnp.float32))
    out_ref[...] = moe_sum + gate * s


def _final(x, moe, w01, sgw, suw, sdw, segw):
    nblk = T // 256
    return pl.pallas_call(
        _final_body,
        grid=(nblk,),
        in_specs=[
            pl.BlockSpec((256, H), lambda i: (i, 0)),
            pl.BlockSpec((256, H // 2), lambda i: (i, 0)),
            pl.BlockSpec((256, H // 2), lambda i: (i + nblk, 0)),
            pl.BlockSpec((256, TOPK), lambda i: (i, 0)),
            pl.BlockSpec((I_SH, H), lambda i: (0, 0)),
            pl.BlockSpec((I_SH, H), lambda i: (0, 0)),
            pl.BlockSpec((H, I_SH), lambda i: (0, 0)),
            pl.BlockSpec((1, H), lambda i: (0, 0)),
        ],
        out_specs=pl.BlockSpec((256, H), lambda i: (i, 0)),
        out_shape=jax.ShapeDtypeStruct((T, H), jnp.float32),
    )(x, moe, moe, w01, sgw, suw, sdw, segw)


def kernel(hidden_states, gate_w, expert_gate_w, expert_up_w, expert_down_w,
           shared_gate_w, shared_up_w, shared_down_w, shared_expert_gate_w):
    Bsz, Sl, Hd = hidden_states.shape
    x = hidden_states.reshape(T, H)

    logits, d_s, d_g, w01 = _router(x, gate_w)

    idx_init = lax.iota(jnp.int32, IDXBUF) % T
    idxbuf = _scatter(d_s.reshape(T * TOPK), idx_init)
    outbuf = None
    xgs = []
    xb16 = x.astype(jnp.bfloat16)
    x16 = lax.bitcast_convert_type(
        jnp.stack([xb16[:, :H // 2], xb16[:, H // 2:]], axis=-1), jnp.int32)
    for g in range(len(GROUPS)):
        nrows = GROUPS[g] * CAP
        per_w = nrows // NW
        chunk = per_w if per_w <= 128 else per_w // ((per_w + 127) // 128)
        xgs.append(_gather_rows(idxbuf, x16, nrows, chunk=chunk,
                                idx_off=GSTART[g] * CAP))
    for g in range(len(GROUPS)):
        outbuf = _experts_group(xgs[g], expert_gate_w, expert_up_w,
                                expert_down_w, outbuf, g)
    moe = _gather_rows(d_g.T.reshape(T * TOPK), outbuf, T * TOPK)

    final = _final(x, moe, w01,
                   shared_gate_w, shared_up_w, shared_down_w,
                   shared_expert_gate_w)
    return final.reshape(Bsz, Sl, Hd), logits
